# baseline (speedup 1.0000x reference)
import functools

import jax
import jax.numpy as jnp
from jax import lax
from jax.experimental import pallas as pl
from jax.experimental.pallas import tpu as pltpu
from jax.experimental.pallas import tpu_sc as plsc

_VOCAB = 100000
_EMBED = 32
_CTX = 20
_BATCH = 1024

_NC = 2
_NS = 16
_NW = _NC * _NS
_BPW = _BATCH // _NW

_LANES = 16


def _gather_sum_sc(inputs, emb_table):
    mesh = plsc.VectorSubcoreMesh(core_axis_name="c", subcore_axis_name="s")

    @functools.partial(
        pl.kernel,
        mesh=mesh,
        out_type=jax.ShapeDtypeStruct((_BATCH, _EMBED), jnp.float32),
        compiler_params=pltpu.CompilerParams(use_tc_tiling_on_sc=False),
        scratch_types=[
            pltpu.VMEM((_CTX, _BPW), jnp.int32),
            pltpu.VMEM((_CTX, _BPW, _EMBED), jnp.float32),
            pltpu.VMEM((_BPW, _EMBED), jnp.float32),
            pltpu.SemaphoreType.DMA,
        ],
    )
    def k(idx_hbm, table_hbm, out_hbm, idx_v, rows_v, acc_v, sem):
        wid = lax.axis_index("s") * _NC + lax.axis_index("c")
        base = wid * _BPW
        idx_copies = [
            pltpu.async_copy(
                idx_hbm.at[pl.ds(c * _BATCH + base, _BPW)], idx_v.at[c], sem
            )
            for c in range(_CTX)
        ]
        for cp in idx_copies:
            cp.wait()
        copies = [
            pltpu.async_copy(table_hbm.at[idx_v.at[c]], rows_v.at[c], sem)
            for c in range(_CTX)
        ]
        for cp in copies:
            cp.wait()

        def body(i, carry):
            for h in range(_EMBED // _LANES):
                sl = pl.ds(h * _LANES, _LANES)
                a = rows_v[0, i, sl]
                for c in range(1, _CTX):
                    a = a + rows_v[c, i, sl]
                acc_v[i, sl] = a
            return carry

        lax.fori_loop(0, _BPW, body, 0)
        pltpu.sync_copy(acc_v, out_hbm.at[pl.ds(base, _BPW)])

    return k(inputs, emb_table)


def _mm_body(s_ref, w_ref, b_ref, o_ref):
    o_ref[...] = (
        lax.dot_general(
            s_ref[...], w_ref[...], (((1,), (1,)), ((), ())),
            preferred_element_type=jnp.float32,
        )
        + b_ref[...]
    )


def _project_tc(summed, W, b2d):
    v_blk = 512
    return pl.pallas_call(
        _mm_body,
        grid=(pl.cdiv(_VOCAB, v_blk),),
        in_specs=[
            pl.BlockSpec((_BATCH, _EMBED), lambda j: (0, 0)),
            pl.BlockSpec((v_blk, _EMBED), lambda j: (j, 0)),
            pl.BlockSpec((1, v_blk), lambda j: (0, j)),
        ],
        out_specs=pl.BlockSpec((_BATCH, v_blk), lambda j: (0, j)),
        out_shape=jax.ShapeDtypeStruct((_BATCH, _VOCAB), jnp.float32),
    )(summed, W, b2d)


def kernel(inputs, emb_table, W, b):
    summed = _gather_sum_sc(inputs.astype(jnp.int32).reshape(-1), emb_table)
    return _project_tc(summed, W, b.reshape(1, _VOCAB))


# speedup vs baseline: 1.1358x; 1.1358x over previous
import functools

import jax
import jax.numpy as jnp
from jax import lax
from jax.experimental import pallas as pl
from jax.experimental.pallas import tpu as pltpu
from jax.experimental.pallas import tpu_sc as plsc

_VOCAB = 100000
_EMBED = 32
_CTX = 20
_BATCH = 1024

_NC = 2
_NS = 16
_NW = _NC * _NS
_BPW = _BATCH // _NW

_LANES = 16


def _gather_sum_sc(inputs, emb_table):
    mesh = plsc.VectorSubcoreMesh(core_axis_name="c", subcore_axis_name="s")

    @functools.partial(
        pl.kernel,
        mesh=mesh,
        out_type=jax.ShapeDtypeStruct((_BATCH, _EMBED), jnp.float32),
        compiler_params=pltpu.CompilerParams(use_tc_tiling_on_sc=False),
        scratch_types=[
            pltpu.VMEM((_CTX, _BPW), jnp.int32),
            pltpu.VMEM((_CTX, _BPW, _EMBED), jnp.float32),
            pltpu.VMEM((_BPW, _EMBED), jnp.float32),
            pltpu.SemaphoreType.DMA,
        ],
    )
    def k(idx_hbm, table_hbm, out_hbm, idx_v, rows_v, acc_v, sem):
        wid = lax.axis_index("s") * _NC + lax.axis_index("c")
        base = wid * _BPW
        idx_copies = [
            pltpu.async_copy(
                idx_hbm.at[c, pl.ds(base, _BPW)], idx_v.at[c], sem
            )
            for c in range(_CTX)
        ]
        for cp in idx_copies:
            cp.wait()
        copies = [
            pltpu.async_copy(table_hbm.at[idx_v.at[c]], rows_v.at[c], sem)
            for c in range(_CTX)
        ]
        for cp in copies:
            cp.wait()

        def body(i, carry):
            for h in range(_EMBED // _LANES):
                sl = pl.ds(h * _LANES, _LANES)
                a = rows_v[0, i, sl]
                for c in range(1, _CTX):
                    a = a + rows_v[c, i, sl]
                acc_v[i, sl] = a
            return carry

        lax.fori_loop(0, _BPW, body, 0)
        pltpu.sync_copy(acc_v, out_hbm.at[pl.ds(base, _BPW)])

    return k(inputs, emb_table)


def _mm_body(s_ref, w_ref, b_ref, o_ref):
    o_ref[...] = (
        lax.dot_general(
            s_ref[...], w_ref[...], (((1,), (1,)), ((), ())),
            preferred_element_type=jnp.float32,
        )
        + b_ref[...]
    )


def _project_tc(summed, W, b2d):
    v_blk = 2048
    return pl.pallas_call(
        _mm_body,
        grid=(pl.cdiv(_VOCAB, v_blk),),
        in_specs=[
            pl.BlockSpec((_BATCH, _EMBED), lambda j: (0, 0)),
            pl.BlockSpec((v_blk, _EMBED), lambda j: (j, 0)),
            pl.BlockSpec((1, v_blk), lambda j: (0, j)),
        ],
        out_specs=pl.BlockSpec((_BATCH, v_blk), lambda j: (0, j)),
        out_shape=jax.ShapeDtypeStruct((_BATCH, _VOCAB), jnp.float32),
    )(summed, W, b2d)


def kernel(inputs, emb_table, W, b):
    summed = _gather_sum_sc(inputs.astype(jnp.int32), emb_table)
    return _project_tc(summed, W, b.reshape(1, _VOCAB))
